# Initial kernel scaffold; baseline (speedup 1.0000x reference)
import jax
import jax.numpy as jnp
from jax.experimental import pallas as pl


def kernel(x, edge_index, edge_weight, batch, W1l, b1l, W1r, W2l, b2l, W2r, W3l, b3l, W3r, W4l, b4l, W4r, Wc, bc):
    raise NotImplementedError("write your pallas kernel here")


# speedup vs baseline: 3.5078x; 3.5078x over previous
import functools

import jax
import jax.numpy as jnp
from jax import lax
from jax.experimental import pallas as pl
from jax.experimental.pallas import tpu as pltpu
from jax.experimental.pallas import tpu_sc as plsc

N = 10000
E = 320000
D = 128
G = 16
NP = 10240
NW = 32
W = 128
EPW = 10112
EP = NW * EPW
NWIN = EPW // W
RPT = NP // 16
R = 1024
GRID = NP // R


def _seg_body(with_cnt, *refs):
    if with_cnt:
        (z_hbm, srcp, dstp, zrows_hbm, zcnt_hbm,
         s_out, cnt_out,
         acc_sh, cnt_sh, sidx, didx, rows, ones_v, cbuf, sem) = refs
    else:
        (z_hbm, srcp, dstp, zrows_hbm,
         s_out,
         acc_sh, sidx, didx, rows, sem) = refs

    cid = lax.axis_index("c")
    sid = lax.axis_index("s")
    wid = sid * 2 + cid
    row0 = sid * RPT

    pltpu.sync_copy(zrows_hbm, rows)
    def zero_body(c, carry):
        pltpu.sync_copy(rows, acc_sh.at[pl.ds(row0 + c * W, W)])
        return carry
    lax.fori_loop(0, RPT // W, zero_body, 0)
    if with_cnt:
        pltpu.sync_copy(zcnt_hbm, cbuf)
        pltpu.sync_copy(cbuf, cnt_sh.at[pl.ds(row0, RPT)])
        for c in range(W // 16):
            ones_v[pl.ds(c * 16, 16)] = jnp.ones((16,), jnp.float32)
    plsc.subcore_barrier()

    def win_body(w, carry):
        base = wid * EPW + w * W
        pltpu.sync_copy(srcp.at[pl.ds(base, W)], sidx)
        pltpu.sync_copy(dstp.at[pl.ds(base, W)], didx)
        pltpu.async_copy(z_hbm.at[sidx], rows, sem).wait()
        pltpu.sync_copy(rows, acc_sh.at[didx], add=True)
        if with_cnt:
            pltpu.sync_copy(ones_v, cnt_sh.at[didx], add=True)
        return carry
    lax.fori_loop(0, NWIN, win_body, 0)
    plsc.subcore_barrier()

    def out_body(c, carry):
        r0 = row0 + c * W
        pltpu.sync_copy(acc_sh.at[pl.ds(r0, W)], rows)
        pltpu.sync_copy(rows, s_out.at[cid, pl.ds(r0, W)])
        return carry
    lax.fori_loop(0, RPT // W, out_body, 0)
    if with_cnt:
        pltpu.sync_copy(cnt_sh.at[pl.ds(row0, RPT)], cbuf)
        pltpu.sync_copy(cbuf, cnt_out.at[cid, pl.ds(row0, RPT)])


@functools.lru_cache(maxsize=None)
def _make_seg_kernel(with_cnt):
    mesh = plsc.VectorSubcoreMesh(core_axis_name="c", subcore_axis_name="s")
    out_type = [jax.ShapeDtypeStruct((2, NP, D), jnp.float32)]
    scratch = [
        pltpu.VMEM_SHARED((NP, D), jnp.float32),
    ]
    if with_cnt:
        out_type.append(jax.ShapeDtypeStruct((2, NP), jnp.float32))
        scratch.append(pltpu.VMEM_SHARED((NP,), jnp.float32))
    scratch += [
        pltpu.VMEM((W,), jnp.int32),
        pltpu.VMEM((W,), jnp.int32),
        pltpu.VMEM((W, D), jnp.float32),
    ]
    if with_cnt:
        scratch += [
            pltpu.VMEM((W,), jnp.float32),
            pltpu.VMEM((RPT,), jnp.float32),
        ]
    scratch.append(pltpu.SemaphoreType.DMA)
    return pl.kernel(
        functools.partial(_seg_body, with_cnt),
        out_type=out_type,
        mesh=mesh,
        scratch_types=scratch,
    )


def _dotT(a, w):
    return lax.dot_general(a, w, (((1,), (1,)), ((), ())),
                           preferred_element_type=jnp.float32)


def _lin_first_body(x_ref, wl_ref, wr_ref, b_ref, z_ref, r_ref):
    x = x_ref[...]
    z_ref[...] = _dotT(x, wl_ref[...])
    r_ref[...] = _dotT(x, wr_ref[...]) + b_ref[...]


def _lin_mid_body(s_ref, cnt_ref, rp_ref, wl_ref, wr_ref, b_ref, z_ref, r_ref):
    inv = 1.0 / jnp.maximum(cnt_ref[0] + cnt_ref[1], 1.0)
    h = jnp.maximum((s_ref[0] + s_ref[1]) * inv + rp_ref[...], 0.0)
    z_ref[...] = _dotT(h, wl_ref[...])
    r_ref[...] = _dotT(h, wr_ref[...]) + b_ref[...]


def _final_body(s_ref, cnt_ref, rp_ref, batch_ref, wc_ref, bc_ref,
                out_ref, gs_acc, gc_acc):
    i = pl.program_id(0)
    inv = 1.0 / jnp.maximum(cnt_ref[0] + cnt_ref[1], 1.0)
    h = jnp.maximum((s_ref[0] + s_ref[1]) * inv + rp_ref[...], 0.0)
    brow = batch_ref[...].reshape(1, R)
    cls = lax.broadcasted_iota(jnp.int32, (G, 1), 0)
    oht = (brow == cls).astype(jnp.float32)

    @pl.when(i == 0)
    def _():
        gs_acc[...] = jnp.zeros_like(gs_acc)
        gc_acc[...] = jnp.zeros_like(gc_acc)

    gs_acc[...] += lax.dot_general(oht, h, (((1,), (0,)), ((), ())),
                                   preferred_element_type=jnp.float32)
    gc_acc[...] += jnp.sum(oht, axis=1, keepdims=True)

    @pl.when(i == GRID - 1)
    def _():
        pooled = gs_acc[...] / jnp.maximum(gc_acc[...], 1.0)
        prod = pooled * wc_ref[...]
        logit = jnp.sum(prod, axis=1, keepdims=True) + bc_ref[0, 0]
        out_ref[...] = jax.nn.sigmoid(logit)


def _full(shape):
    return pl.BlockSpec(shape, lambda i: (0,) * len(shape))


_row_spec = pl.BlockSpec((R, D), lambda i: (i, 0))
_s_spec = pl.BlockSpec((2, R, D), lambda i: (0, i, 0))
_cnt_spec = pl.BlockSpec((2, R, 1), lambda i: (0, i, 0))
_zr_out = [jax.ShapeDtypeStruct((NP, D), jnp.float32),
           jax.ShapeDtypeStruct((NP, D), jnp.float32)]

_lin_first = pl.pallas_call(
    _lin_first_body,
    grid=(GRID,),
    in_specs=[_row_spec, _full((D, D)), _full((D, D)), _full((1, D))],
    out_specs=[_row_spec, _row_spec],
    out_shape=_zr_out,
)

_lin_mid = pl.pallas_call(
    _lin_mid_body,
    grid=(GRID,),
    in_specs=[_s_spec, _cnt_spec, _row_spec,
              _full((D, D)), _full((D, D)), _full((1, D))],
    out_specs=[_row_spec, _row_spec],
    out_shape=_zr_out,
)

_final = pl.pallas_call(
    _final_body,
    grid=(GRID,),
    in_specs=[_s_spec, _cnt_spec, _row_spec,
              pl.BlockSpec((1, 1, R), lambda i: (i, 0, 0)),
              _full((1, D)), _full((1, 1))],
    out_specs=_full((G, 1)),
    out_shape=jax.ShapeDtypeStruct((G, 1), jnp.float32),
    scratch_shapes=[pltpu.VMEM((G, D), jnp.float32),
                    pltpu.VMEM((G, 1), jnp.float32)],
)

def kernel(x, edge_index, edge_weight, batch,
           W1l, b1l, W1r, W2l, b2l, W2r, W3l, b3l, W3r, W4l, b4l, W4r,
           Wc, bc):
    del edge_weight
    src = edge_index[0]
    dst = edge_index[1]
    npad = EP - E
    src_p = jnp.concatenate([src, jnp.zeros((npad,), jnp.int32)])
    dst_p = jnp.concatenate(
        [dst, N + (jnp.arange(npad, dtype=jnp.int32) % (NP - N))])
    batch_p = jnp.concatenate(
        [batch, jnp.full((NP - N,), G, jnp.int32)]).reshape(GRID, 1, R)
    x_p = jnp.pad(x, ((0, NP - N), (0, 0)))
    b1 = b1l.reshape(1, D)
    b2 = b2l.reshape(1, D)
    b3 = b3l.reshape(1, D)
    b4 = b4l.reshape(1, D)
    bc2 = bc.reshape(1, 1)
    zrows = jnp.zeros((W, D), jnp.float32)
    zcnt = jnp.zeros((RPT,), jnp.float32)

    _seg_first = _make_seg_kernel(True)
    _seg_rest = _make_seg_kernel(False)

    z1, r1 = _lin_first(x_p, W1l, W1r, b1)
    s1, cnt = _seg_first(z1, src_p, dst_p, zrows, zcnt)
    cnt = cnt.reshape(2, NP, 1)
    z2, r2 = _lin_mid(s1, cnt, r1, W2l, W2r, b2)
    s2 = _seg_rest(z2, src_p, dst_p, zrows)[0]
    z3, r3 = _lin_mid(s2, cnt, r2, W3l, W3r, b3)
    s3 = _seg_rest(z3, src_p, dst_p, zrows)[0]
    z4, r4 = _lin_mid(s3, cnt, r3, W4l, W4r, b4)
    s4 = _seg_rest(z4, src_p, dst_p, zrows)[0]
    return _final(s4, cnt, r4, batch_p, Wc, bc2)
